# baseline (speedup 1.0000x reference)
import functools

import jax
import jax.numpy as jnp
from jax.experimental import pallas as pl
from jax.experimental.pallas import tpu as pltpu

IN_DIM = 1024
OUT_DIM = 1024
E = 8
TOP_K = 2
TOKENS_PER_BLOCK = 512


def _moe_dense_body(x_ref, gw1_ref, gb1_ref, gw2_ref, gb2_ref,
                    ew_hbm, eb_ref, out_ref, logits_ref, wv_ref, sems):
    i = pl.program_id(0)

    def _copy(e):
        return pltpu.make_async_copy(ew_hbm.at[e], wv_ref.at[e], sems.at[e])

    @pl.when(i == 0)
    def _issue():
        for e in range(E):
            _copy(e).start()

    x = x_ref[...]
    hidden = jnp.dot(x, gw1_ref[...], preferred_element_type=jnp.float32)
    hidden = hidden + gb1_ref[...]
    logits = jnp.dot(hidden, gw2_ref[...], preferred_element_type=jnp.float32)
    logits = logits + gb2_ref[...]
    logits_ref[...] = logits

    m = jnp.max(logits, axis=-1, keepdims=True)
    ex = jnp.exp(logits - m)
    probs = ex / jnp.sum(ex, axis=-1, keepdims=True)

    e_iota = jax.lax.broadcasted_iota(jnp.int32, probs.shape, 1)
    m1 = jnp.max(probs, axis=-1, keepdims=True)
    is1 = (probs == m1)
    a1 = jnp.min(jnp.where(is1, e_iota, E), axis=-1, keepdims=True)
    masked = jnp.where(e_iota == a1, -jnp.inf, probs)
    m2 = jnp.max(masked, axis=-1, keepdims=True)
    is2 = (masked == m2)
    a2 = jnp.min(jnp.where(is2, e_iota, E), axis=-1, keepdims=True)
    denom = m1 + m2
    w1 = m1 / denom
    w2 = m2 / denom
    cw = jnp.where(e_iota == a1, w1, 0.0) + jnp.where(e_iota == a2, w2, 0.0)

    acc = jnp.zeros((x.shape[0], OUT_DIM), dtype=jnp.float32)
    xb = x.astype(jnp.bfloat16)
    for e in range(E):
        @pl.when(i == 0)
        def _wait():
            _copy(e).wait()

        eo = jnp.dot(xb, wv_ref[e].astype(jnp.bfloat16),
                     preferred_element_type=jnp.float32)
        eo = eo + eb_ref[e:e + 1, :]
        acc = acc + eo * cw[:, e:e + 1]
    out_ref[...] = acc


@functools.partial(jax.jit, static_argnames=())
def kernel(x, gate_w1, gate_b1, gate_w2, gate_b2, expert_w, expert_b):
    b, s, h = x.shape
    n = b * s
    flat = x.reshape(n, h)
    grid = (n // TOKENS_PER_BLOCK,)
    out_shapes = (
        jax.ShapeDtypeStruct((n, OUT_DIM), jnp.float32),
        jax.ShapeDtypeStruct((n, E), jnp.float32),
    )
    final, logits = pl.pallas_call(
        _moe_dense_body,
        grid=grid,
        in_specs=[
            pl.BlockSpec((TOKENS_PER_BLOCK, h), lambda i: (i, 0)),
            pl.BlockSpec((h, h // 2), lambda i: (0, 0)),
            pl.BlockSpec((1, h // 2), lambda i: (0, 0)),
            pl.BlockSpec((h // 2, E), lambda i: (0, 0)),
            pl.BlockSpec((1, E), lambda i: (0, 0)),
            pl.BlockSpec(memory_space=pl.ANY),
            pl.BlockSpec((E, OUT_DIM), lambda i: (0, 0)),
        ],
        out_specs=(
            pl.BlockSpec((TOKENS_PER_BLOCK, OUT_DIM), lambda i: (i, 0)),
            pl.BlockSpec((TOKENS_PER_BLOCK, E), lambda i: (i, 0)),
        ),
        out_shape=out_shapes,
        scratch_shapes=[
            pltpu.VMEM((E, h, OUT_DIM), jnp.float32),
            pltpu.SemaphoreType.DMA((E,)),
        ],
        compiler_params=pltpu.CompilerParams(
            dimension_semantics=("arbitrary",),
        ),
    )(flat, gate_w1, gate_b1.reshape(1, -1), gate_w2, gate_b2.reshape(1, -1),
      expert_w, expert_b)
    return final.reshape(b, s, OUT_DIM), logits


# speedup vs baseline: 1.1048x; 1.1048x over previous
import functools

import jax
import jax.numpy as jnp
from jax.experimental import pallas as pl
from jax.experimental.pallas import tpu as pltpu

IN_DIM = 1024
OUT_DIM = 1024
E = 8
TOP_K = 2
TOKENS_PER_BLOCK = 512


def _moe_dense_body(x_ref, gw1_ref, gb1_ref, gw2_ref, gb2_ref,
                    ew_ref, eb_ref, out_ref, logits_ref):
    x = x_ref[...]
    hidden = jnp.dot(x, gw1_ref[...], preferred_element_type=jnp.float32)
    hidden = hidden + gb1_ref[...]
    logits = jnp.dot(hidden, gw2_ref[...], preferred_element_type=jnp.float32)
    logits = logits + gb2_ref[...]
    logits_ref[...] = logits

    m = jnp.max(logits, axis=-1, keepdims=True)
    ex = jnp.exp(logits - m)
    probs = ex / jnp.sum(ex, axis=-1, keepdims=True)

    e_iota = jax.lax.broadcasted_iota(jnp.int32, probs.shape, 1)
    m1 = jnp.max(probs, axis=-1, keepdims=True)
    is1 = (probs == m1)
    a1 = jnp.min(jnp.where(is1, e_iota, E), axis=-1, keepdims=True)
    masked = jnp.where(e_iota == a1, -jnp.inf, probs)
    m2 = jnp.max(masked, axis=-1, keepdims=True)
    is2 = (masked == m2)
    a2 = jnp.min(jnp.where(is2, e_iota, E), axis=-1, keepdims=True)
    denom = m1 + m2
    w1 = m1 / denom
    w2 = m2 / denom
    cw = jnp.where(e_iota == a1, w1, 0.0) + jnp.where(e_iota == a2, w2, 0.0)

    acc = jnp.zeros((x.shape[0], OUT_DIM), dtype=jnp.float32)
    xb = x.astype(jnp.bfloat16)
    for e in range(E):
        eo = jnp.dot(xb, ew_ref[e].astype(jnp.bfloat16),
                     preferred_element_type=jnp.float32)
        eo = eo + eb_ref[e:e + 1, :]
        acc = acc + eo * cw[:, e:e + 1]
    out_ref[...] = acc


@functools.partial(jax.jit, static_argnames=())
def kernel(x, gate_w1, gate_b1, gate_w2, gate_b2, expert_w, expert_b):
    b, s, h = x.shape
    n = b * s
    flat = x.reshape(n, h)
    grid = (n // TOKENS_PER_BLOCK,)
    out_shapes = (
        jax.ShapeDtypeStruct((n, OUT_DIM), jnp.float32),
        jax.ShapeDtypeStruct((n, E), jnp.float32),
    )
    final, logits = pl.pallas_call(
        _moe_dense_body,
        grid=grid,
        in_specs=[
            pl.BlockSpec((TOKENS_PER_BLOCK, h), lambda i: (i, 0)),
            pl.BlockSpec((h, h // 2), lambda i: (0, 0)),
            pl.BlockSpec((1, h // 2), lambda i: (0, 0)),
            pl.BlockSpec((h // 2, E), lambda i: (0, 0)),
            pl.BlockSpec((1, E), lambda i: (0, 0)),
            pl.BlockSpec((E, h, OUT_DIM), lambda i: (0, 0, 0)),
            pl.BlockSpec((E, OUT_DIM), lambda i: (0, 0)),
        ],
        out_specs=(
            pl.BlockSpec((TOKENS_PER_BLOCK, OUT_DIM), lambda i: (i, 0)),
            pl.BlockSpec((TOKENS_PER_BLOCK, E), lambda i: (i, 0)),
        ),
        out_shape=out_shapes,
        compiler_params=pltpu.CompilerParams(
            dimension_semantics=("arbitrary",),
        ),
    )(flat, gate_w1, gate_b1.reshape(1, -1), gate_w2, gate_b2.reshape(1, -1),
      expert_w, expert_b)
    return final.reshape(b, s, OUT_DIM), logits


# speedup vs baseline: 1.1081x; 1.0030x over previous
import functools

import jax
import jax.numpy as jnp
from jax.experimental import pallas as pl
from jax.experimental.pallas import tpu as pltpu

IN_DIM = 1024
OUT_DIM = 1024
E = 8
TOP_K = 2
TOKENS_PER_BLOCK = 512


def _moe_dense_body(x_ref, gw1_ref, gb1_ref, gw2_ref, gb2_ref,
                    ew_ref, eb_ref, out_ref, logits_ref):
    x = x_ref[...]
    hidden = jnp.dot(x, gw1_ref[...], preferred_element_type=jnp.float32)
    hidden = hidden + gb1_ref[...]
    logits = jnp.dot(hidden, gw2_ref[...], preferred_element_type=jnp.float32)
    logits = logits + gb2_ref[...]
    logits_ref[...] = logits

    m = jnp.max(logits, axis=-1, keepdims=True)
    ex = jnp.exp(logits - m)
    probs = ex / jnp.sum(ex, axis=-1, keepdims=True)

    e_iota = jax.lax.broadcasted_iota(jnp.int32, probs.shape, 1)
    m1 = jnp.max(probs, axis=-1, keepdims=True)
    is1 = (probs == m1)
    a1 = jnp.min(jnp.where(is1, e_iota, E), axis=-1, keepdims=True)
    masked = jnp.where(e_iota == a1, -jnp.inf, probs)
    m2 = jnp.max(masked, axis=-1, keepdims=True)
    is2 = (masked == m2)
    a2 = jnp.min(jnp.where(is2, e_iota, E), axis=-1, keepdims=True)
    denom = m1 + m2
    w1 = m1 / denom
    w2 = m2 / denom
    cw = jnp.where(e_iota == a1, w1, 0.0) + jnp.where(e_iota == a2, w2, 0.0)

    acc = jnp.zeros((x.shape[0], OUT_DIM), dtype=jnp.float32)
    xb = x.astype(jnp.bfloat16)
    for e in range(E):
        eo = jnp.dot(xb, ew_ref[e].astype(jnp.bfloat16),
                     preferred_element_type=jnp.float32)
        eo = eo + eb_ref[e:e + 1, :]
        acc = acc + eo * cw[:, e:e + 1]
    out_ref[...] = acc


@functools.partial(jax.jit, static_argnames=())
def kernel(x, gate_w1, gate_b1, gate_w2, gate_b2, expert_w, expert_b):
    b, s, h = x.shape
    n = b * s
    flat = x.reshape(n, h)
    grid = (n // TOKENS_PER_BLOCK,)
    out_shapes = (
        jax.ShapeDtypeStruct((n, OUT_DIM), jnp.float32),
        jax.ShapeDtypeStruct((n, E), jnp.float32),
    )
    final, logits = pl.pallas_call(
        _moe_dense_body,
        grid=grid,
        in_specs=[
            pl.BlockSpec((TOKENS_PER_BLOCK, h), lambda i: (i, 0)),
            pl.BlockSpec((h, h // 2), lambda i: (0, 0)),
            pl.BlockSpec((1, h // 2), lambda i: (0, 0)),
            pl.BlockSpec((h // 2, E), lambda i: (0, 0)),
            pl.BlockSpec((1, E), lambda i: (0, 0)),
            pl.BlockSpec((E, h, OUT_DIM), lambda i: (0, 0, 0)),
            pl.BlockSpec((E, OUT_DIM), lambda i: (0, 0)),
        ],
        out_specs=(
            pl.BlockSpec((TOKENS_PER_BLOCK, OUT_DIM), lambda i: (i, 0)),
            pl.BlockSpec((TOKENS_PER_BLOCK, E), lambda i: (i, 0)),
        ),
        out_shape=out_shapes,
        compiler_params=pltpu.CompilerParams(
            dimension_semantics=("parallel",),
        ),
    )(flat, gate_w1, gate_b1.reshape(1, -1), gate_w2, gate_b2.reshape(1, -1),
      expert_w, expert_b)
    return final.reshape(b, s, OUT_DIM), logits
